# baseline (speedup 1.0000x reference)
import functools

import jax
import jax.numpy as jnp
from jax import lax
from jax.experimental import pallas as pl
from jax.experimental.pallas import tpu as pltpu
from jax.experimental.pallas import tpu_sc as plsc

B, NA, NL, D = 256, 256, 1024, 128
K = 16
NW = 32
BPW = B // NW
BB = 16
MB = 2048

_mesh = plsc.VectorSubcoreMesh(core_axis_name="c", subcore_axis_name="s")


_GDN = lax.GatherDimensionNumbers(offset_dims=(), collapsed_slice_dims=(0,),
                                  start_index_map=(0,))


def _vgather(v, idx):
    return lax.gather(v, idx[:, None], dimension_numbers=_GDN,
                      slice_sizes=(1,),
                      mode=lax.GatherScatterMode.PROMISE_IN_BOUNDS)


def _merge_top(tk, ti, ks, is_, smallest):
    kr = lax.rev(ks, (0,))
    ir = lax.rev(is_, (0,))
    if smallest:
        cond = (tk < kr) | ((tk == kr) & (ti < ir))
    else:
        cond = (tk > kr) | ((tk == kr) & (ti < ir))
    mk = jnp.where(cond, tk, kr)
    mi = jnp.where(cond, ti, ir)
    return plsc.sort_key_val(mk, mi, descending=not smallest)


@functools.partial(
    pl.kernel,
    out_type=(jax.ShapeDtypeStruct((B, K), jnp.int32),
              jax.ShapeDtypeStruct((B, K), jnp.int32),
              jax.ShapeDtypeStruct((B, 2 * K, D), jnp.float32)),
    mesh=_mesh,
    compiler_params=pltpu.CompilerParams(needs_layout_passes=False),
    scratch_types=[
        pltpu.VMEM((NA,), jnp.float32),
        pltpu.VMEM((NA,), jnp.float32),
        pltpu.VMEM((NA,), jnp.float32),
        pltpu.VMEM((NL,), jnp.float32),
        pltpu.VMEM((NL,), jnp.float32),
        pltpu.VMEM((NL,), jnp.float32),
        pltpu.VMEM((K,), jnp.int32),
        pltpu.VMEM((K,), jnp.int32),
        pltpu.VMEM((K,), jnp.int32),
        pltpu.VMEM((K, D), jnp.float32),
        pltpu.SemaphoreType.DMA,
    ],
)
def _sc_select_gather(mspike, axh, ayh, lcxh, lcyh, lmaskh, afh, lfh,
                      aidxh, lidxh, nodesh,
                      sp_v, ax_v, ay_v, lx_v, ly_v, lm_v,
                      ai_v, li_v, fl_v, rows_v, sem):
    wid = lax.axis_index("s") * 2 + lax.axis_index("c")
    iota = jnp.arange(K, dtype=jnp.int32)

    def one_batch(k, _):
        b = wid * BPW + k
        pltpu.sync_copy(mspike.at[b], sp_v)
        pltpu.sync_copy(axh.at[b], ax_v)
        pltpu.sync_copy(ayh.at[b], ay_v)
        pltpu.sync_copy(lcxh.at[b], lx_v)
        pltpu.sync_copy(lcyh.at[b], ly_v)
        pltpu.sync_copy(lmaskh.at[b], lm_v)

        tk, ti = plsc.sort_key_val(sp_v[pl.ds(0, K)], iota, descending=True)
        for c in range(1, NA // K):
            ks, is_ = plsc.sort_key_val(sp_v[pl.ds(c * K, K)], iota + c * K,
                                        descending=True)
            tk, ti = _merge_top(tk, ti, ks, is_, smallest=False)
        ai_v[...] = ti
        pltpu.sync_copy(ai_v, aidxh.at[b])

        acx = plsc.load_gather(ax_v, [ti])
        acy = plsc.load_gather(ay_v, [ti])
        axt = [_vgather(acx, jnp.full((K,), t, jnp.int32)) for t in range(K)]
        ayt = [_vgather(acy, jnp.full((K,), t, jnp.int32)) for t in range(K)]

        def lane_chunk(c, carry):
            ids = iota + c * K
            lx = plsc.load_gather(lx_v, [ids])
            ly = plsc.load_gather(ly_v, [ids])
            best = jnp.full((K,), jnp.inf, dtype=jnp.float32)
            for t in range(K):
                dx = lx - axt[t]
                dy = ly - ayt[t]
                best = jnp.minimum(best, dx * dx + dy * dy)
            best = best + plsc.load_gather(lm_v, [ids])
            ks, is_ = plsc.sort_key_val(best, ids, descending=False)
            tk, ti = carry
            return _merge_top(tk, ti, ks, is_, smallest=True)

        carry = (jnp.full((K,), jnp.inf, dtype=jnp.float32),
                 jnp.full((K,), NL, dtype=jnp.int32))
        for c in range(NL // K):
            carry = lane_chunk(c, carry)
        ltk, lti = carry
        li_v[...] = lti
        pltpu.sync_copy(li_v, lidxh.at[b])

        fl_v[...] = ti + b * NA
        pltpu.async_copy(afh.at[fl_v], rows_v, sem).wait()
        pltpu.sync_copy(rows_v, nodesh.at[b, pl.ds(0, K)])
        fl_v[...] = lti + b * NL
        pltpu.async_copy(lfh.at[fl_v], rows_v, sem).wait()
        pltpu.sync_copy(rows_v, nodesh.at[b, pl.ds(K, K)])
        return _

    lax.fori_loop(0, BPW, one_batch, None)


@functools.partial(
    pl.kernel,
    out_type=(),
    mesh=_mesh,
    compiler_params=pltpu.CompilerParams(needs_layout_passes=False),
    scratch_types=[
        pltpu.VMEM((K,), jnp.int32),
        pltpu.VMEM((K,), jnp.int32),
        pltpu.VMEM((K, D), jnp.float32),
        pltpu.SemaphoreType.DMA,
    ],
)
def _sc_scatter(aref, lref, updh, aidxh, lidxh,
                ai_v, fl_v, urows_v, sem):
    wid = lax.axis_index("s") * 2 + lax.axis_index("c")

    def one_batch(k, _):
        b = wid * BPW + k
        pltpu.sync_copy(aidxh.at[b], ai_v)
        fl_v[...] = ai_v[...] + b * NA
        pltpu.sync_copy(updh.at[b, pl.ds(0, K)], urows_v)
        pltpu.async_copy(urows_v, aref.at[fl_v], sem).wait()
        pltpu.sync_copy(lidxh.at[b], ai_v)
        fl_v[...] = ai_v[...] + b * NL
        pltpu.sync_copy(updh.at[b, pl.ds(K, K)], urows_v)
        pltpu.async_copy(urows_v, lref.at[fl_v], sem).wait()
        return _

    lax.fori_loop(0, BPW, one_batch, None)


def _copy_body(actor_in, lane_in, actor_out, lane_out):
    actor_out[...] = actor_in[...]
    lane_out[...] = lane_in[...]


def _mlp_body(nodes_in, W1_0, b1_0, W2_0, b2_0, W1_1, b1_1, W2_1, b2_1,
              ln_g, ln_b, out_ref):
    nodes = nodes_in[...]
    params = [(W1_0, b1_0, W2_0, b2_0), (W1_1, b1_1, W2_1, b2_1)]
    gv = ln_g[...]
    bv = ln_b[...]
    for (W1, b1, W2, b2) in params:
        h = lax.dot_general(nodes, W1[...], (((1,), (1,)), ((), ())),
                            preferred_element_type=jnp.float32) + b1[...]
        h = h * 0.5 * (1.0 + lax.erf(h * 0.7071067811865476))
        h = lax.dot_general(h, W2[...], (((1,), (1,)), ((), ())),
                            preferred_element_type=jnp.float32) + b2[...]
        x = nodes + h
        mu = jnp.mean(x, axis=-1, keepdims=True)
        var = jnp.mean((x - mu) * (x - mu), axis=-1, keepdims=True)
        nodes = (x - mu) / jnp.sqrt(var + 1e-5) * gv + bv
    out_ref[...] = nodes


def kernel(actor_feat, lane_feat, lane_centers, lane_key_valid_mask, x_centers,
           x_key_valid_mask, spike_rate, W1_0, b1_0, W2_0, b2_0, W1_1, b1_1,
           W2_1, b2_1, ln_g, ln_b):
    f32 = jnp.float32
    mspike = jnp.where(x_key_valid_mask, spike_rate, -jnp.inf)
    lmask = jnp.where(lane_key_valid_mask, 0.0, jnp.inf).astype(f32)
    ax = x_centers[:, :, 0]
    ay = x_centers[:, :, 1]
    lcx = lane_centers[:, :, 0]
    lcy = lane_centers[:, :, 1]
    af = actor_feat.reshape(B * NA, D)
    lf = lane_feat.reshape(B * NL, D)

    aidx, lidx, nodes = _sc_select_gather(mspike, ax, ay, lcx, lcy, lmask,
                                          af, lf)

    actor_spec = pl.BlockSpec((BB, NA, D), lambda i: (i, 0, 0))
    lane_spec = pl.BlockSpec((BB, NL, D), lambda i: (i, 0, 0))
    actor_copy, lane_copy = pl.pallas_call(
        _copy_body,
        grid=(B // BB,),
        in_specs=[actor_spec, lane_spec],
        out_specs=[actor_spec, lane_spec],
        out_shape=[jax.ShapeDtypeStruct((B, NA, D), f32),
                   jax.ShapeDtypeStruct((B, NL, D), f32)],
    )(actor_feat, lane_feat)

    w_spec = pl.BlockSpec((D, D), lambda i: (0, 0))
    v_spec = pl.BlockSpec((1, D), lambda i: (0, 0))
    n_spec = pl.BlockSpec((MB, D), lambda i: (i, 0))
    upd = pl.pallas_call(
        _mlp_body,
        grid=(B * 2 * K // MB,),
        in_specs=[n_spec, w_spec, v_spec, w_spec, v_spec,
                  w_spec, v_spec, w_spec, v_spec, v_spec, v_spec],
        out_specs=n_spec,
        out_shape=jax.ShapeDtypeStruct((B * 2 * K, D), f32),
    )(nodes.reshape(B * 2 * K, D),
      W1_0, b1_0.reshape(1, D), W2_0, b2_0.reshape(1, D),
      W1_1, b1_1.reshape(1, D), W2_1, b2_1.reshape(1, D),
      ln_g.reshape(1, D), ln_b.reshape(1, D))

    aref = jax.new_ref(actor_copy.reshape(B * NA, D))
    lref = jax.new_ref(lane_copy.reshape(B * NL, D))
    _sc_scatter(aref, lref, upd.reshape(B, 2 * K, D), aidx, lidx)
    actor_out = jax.freeze(aref).reshape(B, NA, D)
    lane_out = jax.freeze(lref).reshape(B, NL, D)
    return (actor_out, lane_out)


# speedup vs baseline: 1.2563x; 1.2563x over previous
import functools

import jax
import jax.numpy as jnp
from jax import lax
from jax.experimental import pallas as pl
from jax.experimental.pallas import tpu as pltpu
from jax.experimental.pallas import tpu_sc as plsc

B, NA, NL, D = 256, 256, 1024, 128
K = 16
NW = 32
BPW = B // NW
BB = 16
MB = 2048

_mesh = plsc.VectorSubcoreMesh(core_axis_name="c", subcore_axis_name="s")
_GDN = lax.GatherDimensionNumbers(offset_dims=(), collapsed_slice_dims=(0,),
                                  start_index_map=(0,))


def _vgather(v, idx):
    return lax.gather(v, idx[:, None], dimension_numbers=_GDN,
                      slice_sizes=(1,),
                      mode=lax.GatherScatterMode.PROMISE_IN_BOUNDS)


def _merge_top(tk, ti, ks, is_, smallest):
    kr = lax.rev(ks, (0,))
    ir = lax.rev(is_, (0,))
    if smallest:
        cond = (tk < kr) | ((tk == kr) & (ti < ir))
    else:
        cond = (tk > kr) | ((tk == kr) & (ti < ir))
    mk = jnp.where(cond, tk, kr)
    mi = jnp.where(cond, ti, ir)
    sk, si = plsc.sort_key_val(mk, mi, descending=not smallest)
    return sk, si


@functools.partial(
    pl.kernel,
    out_type=(jax.ShapeDtypeStruct((B, K), jnp.int32),
              jax.ShapeDtypeStruct((B, K), jnp.int32),
              jax.ShapeDtypeStruct((B * K, D), jnp.float32),
              jax.ShapeDtypeStruct((B * K, D), jnp.float32)),
    mesh=_mesh,
    compiler_params=pltpu.CompilerParams(needs_layout_passes=False),
    scratch_types=[
        pltpu.VMEM((BPW, NA), jnp.float32),
        pltpu.VMEM((BPW, NA), jnp.float32),
        pltpu.VMEM((BPW, NA), jnp.float32),
        pltpu.VMEM((BPW, NL), jnp.float32),
        pltpu.VMEM((BPW, NL), jnp.float32),
        pltpu.VMEM((BPW, NL), jnp.float32),
        pltpu.VMEM((BPW, K), jnp.int32),
        pltpu.VMEM((BPW, K), jnp.int32),
        pltpu.VMEM((BPW * K,), jnp.int32),
        pltpu.VMEM((BPW * K,), jnp.int32),
        pltpu.VMEM((BPW * K, D), jnp.float32),
        pltpu.SemaphoreType.DMA,
    ],
)
def _sc_select_gather(mspike, axh, ayh, lcxh, lcyh, lmaskh, afh, lfh,
                      aidxh, lidxh, anodesh, lnodesh,
                      sp_v, ax_v, ay_v, lx_v, ly_v, lm_v,
                      ai_v, li_v, afl_v, lfl_v, rows_v, sem):
    wid = lax.axis_index("s") * 2 + lax.axis_index("c")
    b0 = wid * BPW
    iota = jnp.arange(K, dtype=jnp.int32)

    pltpu.sync_copy(mspike.at[pl.ds(b0, BPW)], sp_v)
    pltpu.sync_copy(axh.at[pl.ds(b0, BPW)], ax_v)
    pltpu.sync_copy(ayh.at[pl.ds(b0, BPW)], ay_v)
    pltpu.sync_copy(lcxh.at[pl.ds(b0, BPW)], lx_v)
    pltpu.sync_copy(lcyh.at[pl.ds(b0, BPW)], ly_v)
    pltpu.sync_copy(lmaskh.at[pl.ds(b0, BPW)], lm_v)

    def one_batch(k, _):
        kf = jnp.full((K,), k, jnp.int32)

        tk, ti = plsc.sort_key_val(plsc.load_gather(sp_v, [kf, iota]), iota,
                                   descending=True)
        for c in range(1, NA // K):
            kc = plsc.load_gather(sp_v, [kf, iota + c * K])
            t16 = _vgather(tk, jnp.full((K,), 15, jnp.int32))

            def amerge(tk, ti, kc=kc, c=c):
                ks, is_ = plsc.sort_key_val(kc, iota + c * K, descending=True)
                return _merge_top(tk, ti, ks, is_, smallest=False)

            tk, ti = lax.cond(jnp.any(kc >= t16), amerge,
                              lambda tk, ti: (tk, ti), tk, ti)
        plsc.store_scatter(ai_v, [kf, iota], ti)
        plsc.store_scatter(afl_v, [iota + k * K], ti + (b0 + k) * NA)

        acx = plsc.load_gather(ax_v, [kf, ti])
        acy = plsc.load_gather(ay_v, [kf, ti])
        axt = [_vgather(acx, jnp.full((K,), t, jnp.int32)) for t in range(K)]
        ayt = [_vgather(acy, jnp.full((K,), t, jnp.int32)) for t in range(K)]

        tk = jnp.full((K,), jnp.inf, dtype=jnp.float32)
        ti = jnp.full((K,), NL, dtype=jnp.int32)
        for c in range(NL // K):
            ids = iota + c * K
            lx = plsc.load_gather(lx_v, [kf, ids])
            ly = plsc.load_gather(ly_v, [kf, ids])
            best = jnp.full((K,), jnp.inf, dtype=jnp.float32)
            for t in range(K):
                dx = lx - axt[t]
                dy = ly - ayt[t]
                best = jnp.minimum(best, dx * dx + dy * dy)
            best = best + plsc.load_gather(lm_v, [kf, ids])
            t16 = _vgather(tk, jnp.full((K,), 15, jnp.int32))

            def lmerge(tk, ti, best=best, ids=ids):
                ks, is_ = plsc.sort_key_val(best, ids, descending=False)
                return _merge_top(tk, ti, ks, is_, smallest=True)

            tk, ti = lax.cond(jnp.any(best <= t16), lmerge,
                              lambda tk, ti: (tk, ti), tk, ti)
        plsc.store_scatter(li_v, [kf, iota], ti)
        plsc.store_scatter(lfl_v, [iota + k * K], ti + (b0 + k) * NL)
        return _

    lax.fori_loop(0, BPW, one_batch, None)

    pltpu.sync_copy(ai_v, aidxh.at[pl.ds(b0, BPW)])
    pltpu.sync_copy(li_v, lidxh.at[pl.ds(b0, BPW)])
    pltpu.async_copy(afh.at[afl_v], rows_v, sem).wait()
    pltpu.sync_copy(rows_v, anodesh.at[pl.ds(b0 * K, BPW * K)])
    pltpu.async_copy(lfh.at[lfl_v], rows_v, sem).wait()
    pltpu.sync_copy(rows_v, lnodesh.at[pl.ds(b0 * K, BPW * K)])


@functools.partial(
    pl.kernel,
    out_type=(),
    mesh=_mesh,
    compiler_params=pltpu.CompilerParams(needs_layout_passes=False),
    scratch_types=[
        pltpu.VMEM((BPW, K), jnp.int32),
        pltpu.VMEM((BPW * K,), jnp.int32),
        pltpu.VMEM((BPW * K, D), jnp.float32),
        pltpu.SemaphoreType.DMA,
    ],
)
def _sc_scatter(aref, lref, updah, updlh, aidxh, lidxh,
                ai_v, fl_v, urows_v, sem):
    wid = lax.axis_index("s") * 2 + lax.axis_index("c")
    b0 = wid * BPW
    iota = jnp.arange(K, dtype=jnp.int32)

    def flatten(idxh, stride):
        pltpu.sync_copy(idxh.at[pl.ds(b0, BPW)], ai_v)

        def one(k, _):
            kf = jnp.full((K,), k, jnp.int32)
            ids = plsc.load_gather(ai_v, [kf, iota])
            plsc.store_scatter(fl_v, [iota + k * K], ids + (b0 + k) * stride)
            return _

        lax.fori_loop(0, BPW, one, None)

    flatten(aidxh, NA)
    pltpu.sync_copy(updah.at[pl.ds(b0 * K, BPW * K)], urows_v)
    pltpu.async_copy(urows_v, aref.at[fl_v], sem).wait()
    flatten(lidxh, NL)
    pltpu.sync_copy(updlh.at[pl.ds(b0 * K, BPW * K)], urows_v)
    pltpu.async_copy(urows_v, lref.at[fl_v], sem).wait()


def _copy_body(actor_in, lane_in, actor_out, lane_out):
    actor_out[...] = actor_in[...]
    lane_out[...] = lane_in[...]


def _mlp(nodes, params, gv, bv):
    for (W1, b1, W2, b2) in params:
        h = lax.dot_general(nodes, W1, (((1,), (1,)), ((), ())),
                            preferred_element_type=jnp.float32) + b1
        h = h * 0.5 * (1.0 + lax.erf(h * 0.7071067811865476))
        h = lax.dot_general(h, W2, (((1,), (1,)), ((), ())),
                            preferred_element_type=jnp.float32) + b2
        x = nodes + h
        mu = jnp.mean(x, axis=-1, keepdims=True)
        var = jnp.mean((x - mu) * (x - mu), axis=-1, keepdims=True)
        nodes = (x - mu) / jnp.sqrt(var + 1e-5) * gv + bv
    return nodes


def _mlp_body(an_in, ln_in, W1_0, b1_0, W2_0, b2_0, W1_1, b1_1, W2_1, b2_1,
              ln_g, ln_b, an_out, ln_out):
    params = [(W1_0[...], b1_0[...], W2_0[...], b2_0[...]),
              (W1_1[...], b1_1[...], W2_1[...], b2_1[...])]
    gv = ln_g[...]
    bv = ln_b[...]
    an_out[...] = _mlp(an_in[...], params, gv, bv)
    ln_out[...] = _mlp(ln_in[...], params, gv, bv)


def kernel(actor_feat, lane_feat, lane_centers, lane_key_valid_mask, x_centers,
           x_key_valid_mask, spike_rate, W1_0, b1_0, W2_0, b2_0, W1_1, b1_1,
           W2_1, b2_1, ln_g, ln_b):
    f32 = jnp.float32
    mspike = jnp.where(x_key_valid_mask, spike_rate, -jnp.inf)
    lmask = jnp.where(lane_key_valid_mask, 0.0, jnp.inf).astype(f32)
    ax = x_centers[:, :, 0]
    ay = x_centers[:, :, 1]
    lcx = lane_centers[:, :, 0]
    lcy = lane_centers[:, :, 1]
    af = actor_feat.reshape(B * NA, D)
    lf = lane_feat.reshape(B * NL, D)

    aidx, lidx, anodes, lnodes = _sc_select_gather(mspike, ax, ay, lcx, lcy,
                                                   lmask, af, lf)

    actor_spec = pl.BlockSpec((BB, NA, D), lambda i: (i, 0, 0))
    lane_spec = pl.BlockSpec((BB, NL, D), lambda i: (i, 0, 0))
    actor_copy, lane_copy = pl.pallas_call(
        _copy_body,
        grid=(B // BB,),
        in_specs=[actor_spec, lane_spec],
        out_specs=[actor_spec, lane_spec],
        out_shape=[jax.ShapeDtypeStruct((B, NA, D), f32),
                   jax.ShapeDtypeStruct((B, NL, D), f32)],
    )(actor_feat, lane_feat)

    w_spec = pl.BlockSpec((D, D), lambda i: (0, 0))
    v_spec = pl.BlockSpec((1, D), lambda i: (0, 0))
    n_spec = pl.BlockSpec((MB, D), lambda i: (i, 0))
    upda, updl = pl.pallas_call(
        _mlp_body,
        grid=(B * K // MB,),
        in_specs=[n_spec, n_spec, w_spec, v_spec, w_spec, v_spec,
                  w_spec, v_spec, w_spec, v_spec, v_spec, v_spec],
        out_specs=[n_spec, n_spec],
        out_shape=[jax.ShapeDtypeStruct((B * K, D), f32),
                   jax.ShapeDtypeStruct((B * K, D), f32)],
    )(anodes, lnodes,
      W1_0, b1_0.reshape(1, D), W2_0, b2_0.reshape(1, D),
      W1_1, b1_1.reshape(1, D), W2_1, b2_1.reshape(1, D),
      ln_g.reshape(1, D), ln_b.reshape(1, D))

    aref = jax.new_ref(actor_copy.reshape(B * NA, D))
    lref = jax.new_ref(lane_copy.reshape(B * NL, D))
    _sc_scatter(aref, lref, upda, updl, aidx, lidx)
    actor_out = jax.freeze(aref).reshape(B, NA, D)
    lane_out = jax.freeze(lref).reshape(B, NL, D)
    return (actor_out, lane_out)


# speedup vs baseline: 1.2588x; 1.0021x over previous
import functools

import jax
import jax.numpy as jnp
from jax import lax
from jax.experimental import pallas as pl
from jax.experimental.pallas import tpu as pltpu
from jax.experimental.pallas import tpu_sc as plsc

B, NA, NL, D = 256, 256, 1024, 128
K = 16
NW = 32
BPW = B // NW
BB = 16
MB = 2048

_mesh = plsc.VectorSubcoreMesh(core_axis_name="c", subcore_axis_name="s")
_GDN = lax.GatherDimensionNumbers(offset_dims=(), collapsed_slice_dims=(0,),
                                  start_index_map=(0,))


def _vgather(v, idx):
    return lax.gather(v, idx[:, None], dimension_numbers=_GDN,
                      slice_sizes=(1,),
                      mode=lax.GatherScatterMode.PROMISE_IN_BOUNDS)


def _merge_top(tk, ti, ks, is_, smallest):
    kr = lax.rev(ks, (0,))
    ir = lax.rev(is_, (0,))
    if smallest:
        cond = (tk < kr) | ((tk == kr) & (ti < ir))
    else:
        cond = (tk > kr) | ((tk == kr) & (ti < ir))
    mk = jnp.where(cond, tk, kr)
    mi = jnp.where(cond, ti, ir)
    sk, si = plsc.sort_key_val(mk, mi, descending=not smallest)
    return sk, si


@functools.partial(
    pl.kernel,
    out_type=(jax.ShapeDtypeStruct((B, K), jnp.int32),
              jax.ShapeDtypeStruct((B, K), jnp.int32),
              jax.ShapeDtypeStruct((B * K, D), jnp.float32),
              jax.ShapeDtypeStruct((B * K, D), jnp.float32)),
    mesh=_mesh,
    compiler_params=pltpu.CompilerParams(needs_layout_passes=False),
    scratch_types=[
        pltpu.VMEM((BPW, NA), jnp.float32),
        pltpu.VMEM((BPW, NA), jnp.float32),
        pltpu.VMEM((BPW, NA), jnp.float32),
        pltpu.VMEM((BPW, NL), jnp.float32),
        pltpu.VMEM((BPW, NL), jnp.float32),
        pltpu.VMEM((BPW, NL), jnp.float32),
        pltpu.VMEM((BPW, K), jnp.int32),
        pltpu.VMEM((BPW, K), jnp.int32),
        pltpu.VMEM((BPW * K,), jnp.int32),
        pltpu.VMEM((BPW * K,), jnp.int32),
        pltpu.VMEM((BPW * K, D), jnp.float32),
        pltpu.SemaphoreType.DMA,
    ],
)
def _sc_select_gather(mspike, axh, ayh, lcxh, lcyh, lmaskh, afh, lfh,
                      aidxh, lidxh, anodesh, lnodesh,
                      sp_v, ax_v, ay_v, lx_v, ly_v, lm_v,
                      ai_v, li_v, afl_v, lfl_v, rows_v, sem):
    wid = lax.axis_index("s") * 2 + lax.axis_index("c")
    b0 = wid * BPW
    iota = jnp.arange(K, dtype=jnp.int32)

    pltpu.sync_copy(mspike.at[pl.ds(b0, BPW)], sp_v)
    pltpu.sync_copy(axh.at[pl.ds(b0, BPW)], ax_v)
    pltpu.sync_copy(ayh.at[pl.ds(b0, BPW)], ay_v)
    pltpu.sync_copy(lcxh.at[pl.ds(b0, BPW)], lx_v)
    pltpu.sync_copy(lcyh.at[pl.ds(b0, BPW)], ly_v)
    pltpu.sync_copy(lmaskh.at[pl.ds(b0, BPW)], lm_v)

    def one_batch(k, _):
        kf = jnp.full((K,), k, jnp.int32)

        tk, ti = plsc.sort_key_val(plsc.load_gather(sp_v, [kf, iota]), iota,
                                   descending=True)
        for c in range(1, NA // K):
            kc = plsc.load_gather(sp_v, [kf, iota + c * K])
            t16 = _vgather(tk, jnp.full((K,), 15, jnp.int32))

            def amerge(tk, ti, kc=kc, c=c):
                ks, is_ = plsc.sort_key_val(kc, iota + c * K, descending=True)
                return _merge_top(tk, ti, ks, is_, smallest=False)

            tk, ti = lax.cond(jnp.any(kc >= t16), amerge,
                              lambda tk, ti: (tk, ti), tk, ti)
        plsc.store_scatter(ai_v, [kf, iota], ti)
        plsc.store_scatter(afl_v, [iota + k * K], ti + (b0 + k) * NA)

        acx = plsc.load_gather(ax_v, [kf, ti])
        acy = plsc.load_gather(ay_v, [kf, ti])
        axt = [_vgather(acx, jnp.full((K,), t, jnp.int32)) for t in range(K)]
        ayt = [_vgather(acy, jnp.full((K,), t, jnp.int32)) for t in range(K)]

        def lane_chunk(c, carry):
            ids = iota + c * K
            lx = plsc.load_gather(lx_v, [kf, ids])
            ly = plsc.load_gather(ly_v, [kf, ids])
            best = jnp.full((K,), jnp.inf, dtype=jnp.float32)
            for t in range(K):
                dx = lx - axt[t]
                dy = ly - ayt[t]
                best = jnp.minimum(best, dx * dx + dy * dy)
            best = best + plsc.load_gather(lm_v, [kf, ids])
            tk, ti = carry
            t16 = _vgather(tk, jnp.full((K,), 15, jnp.int32))

            def lmerge(tk, ti, best=best, ids=ids):
                ks, is_ = plsc.sort_key_val(best, ids, descending=False)
                return _merge_top(tk, ti, ks, is_, smallest=True)

            return lax.cond(jnp.any(best <= t16), lmerge,
                            lambda tk, ti: (tk, ti), tk, ti)

        tk, ti = lax.fori_loop(0, NL // K, lane_chunk,
                               (jnp.full((K,), jnp.inf, dtype=jnp.float32),
                                jnp.full((K,), NL, dtype=jnp.int32)))
        plsc.store_scatter(li_v, [kf, iota], ti)
        plsc.store_scatter(lfl_v, [iota + k * K], ti + (b0 + k) * NL)
        return _

    lax.fori_loop(0, BPW, one_batch, None)

    pltpu.sync_copy(ai_v, aidxh.at[pl.ds(b0, BPW)])
    pltpu.sync_copy(li_v, lidxh.at[pl.ds(b0, BPW)])
    pltpu.async_copy(afh.at[afl_v], rows_v, sem).wait()
    pltpu.sync_copy(rows_v, anodesh.at[pl.ds(b0 * K, BPW * K)])
    pltpu.async_copy(lfh.at[lfl_v], rows_v, sem).wait()
    pltpu.sync_copy(rows_v, lnodesh.at[pl.ds(b0 * K, BPW * K)])


@functools.partial(
    pl.kernel,
    out_type=(),
    mesh=_mesh,
    compiler_params=pltpu.CompilerParams(needs_layout_passes=False),
    scratch_types=[
        pltpu.VMEM((BPW, K), jnp.int32),
        pltpu.VMEM((BPW * K,), jnp.int32),
        pltpu.VMEM((BPW * K, D), jnp.float32),
        pltpu.SemaphoreType.DMA,
    ],
)
def _sc_scatter(aref, lref, updah, updlh, aidxh, lidxh,
                ai_v, fl_v, urows_v, sem):
    wid = lax.axis_index("s") * 2 + lax.axis_index("c")
    b0 = wid * BPW
    iota = jnp.arange(K, dtype=jnp.int32)

    def flatten(idxh, stride):
        pltpu.sync_copy(idxh.at[pl.ds(b0, BPW)], ai_v)

        def one(k, _):
            kf = jnp.full((K,), k, jnp.int32)
            ids = plsc.load_gather(ai_v, [kf, iota])
            plsc.store_scatter(fl_v, [iota + k * K], ids + (b0 + k) * stride)
            return _

        lax.fori_loop(0, BPW, one, None)

    flatten(aidxh, NA)
    pltpu.sync_copy(updah.at[pl.ds(b0 * K, BPW * K)], urows_v)
    pltpu.async_copy(urows_v, aref.at[fl_v], sem).wait()
    flatten(lidxh, NL)
    pltpu.sync_copy(updlh.at[pl.ds(b0 * K, BPW * K)], urows_v)
    pltpu.async_copy(urows_v, lref.at[fl_v], sem).wait()


def _copy_body(actor_in, lane_in, actor_out, lane_out):
    actor_out[...] = actor_in[...]
    lane_out[...] = lane_in[...]


def _mlp(nodes, params, gv, bv):
    for (W1, b1, W2, b2) in params:
        h = lax.dot_general(nodes, W1, (((1,), (1,)), ((), ())),
                            preferred_element_type=jnp.float32) + b1
        h = h * 0.5 * (1.0 + lax.erf(h * 0.7071067811865476))
        h = lax.dot_general(h, W2, (((1,), (1,)), ((), ())),
                            preferred_element_type=jnp.float32) + b2
        x = nodes + h
        mu = jnp.mean(x, axis=-1, keepdims=True)
        var = jnp.mean((x - mu) * (x - mu), axis=-1, keepdims=True)
        nodes = (x - mu) / jnp.sqrt(var + 1e-5) * gv + bv
    return nodes


def _mlp_body(an_in, ln_in, W1_0, b1_0, W2_0, b2_0, W1_1, b1_1, W2_1, b2_1,
              ln_g, ln_b, an_out, ln_out):
    params = [(W1_0[...], b1_0[...], W2_0[...], b2_0[...]),
              (W1_1[...], b1_1[...], W2_1[...], b2_1[...])]
    gv = ln_g[...]
    bv = ln_b[...]
    an_out[...] = _mlp(an_in[...], params, gv, bv)
    ln_out[...] = _mlp(ln_in[...], params, gv, bv)


def kernel(actor_feat, lane_feat, lane_centers, lane_key_valid_mask, x_centers,
           x_key_valid_mask, spike_rate, W1_0, b1_0, W2_0, b2_0, W1_1, b1_1,
           W2_1, b2_1, ln_g, ln_b):
    f32 = jnp.float32
    mspike = jnp.where(x_key_valid_mask, spike_rate, -jnp.inf)
    lmask = jnp.where(lane_key_valid_mask, 0.0, jnp.inf).astype(f32)
    ax = x_centers[:, :, 0]
    ay = x_centers[:, :, 1]
    lcx = lane_centers[:, :, 0]
    lcy = lane_centers[:, :, 1]
    af = actor_feat.reshape(B * NA, D)
    lf = lane_feat.reshape(B * NL, D)

    actor_spec = pl.BlockSpec((BB, NA, D), lambda i: (i, 0, 0))
    lane_spec = pl.BlockSpec((BB, NL, D), lambda i: (i, 0, 0))
    actor_copy, lane_copy = pl.pallas_call(
        _copy_body,
        grid=(B // BB,),
        in_specs=[actor_spec, lane_spec],
        out_specs=[actor_spec, lane_spec],
        out_shape=[jax.ShapeDtypeStruct((B, NA, D), f32),
                   jax.ShapeDtypeStruct((B, NL, D), f32)],
    )(actor_feat, lane_feat)

    aidx, lidx, anodes, lnodes = _sc_select_gather(mspike, ax, ay, lcx, lcy,
                                                   lmask, af, lf)

    w_spec = pl.BlockSpec((D, D), lambda i: (0, 0))
    v_spec = pl.BlockSpec((1, D), lambda i: (0, 0))
    n_spec = pl.BlockSpec((MB, D), lambda i: (i, 0))
    upda, updl = pl.pallas_call(
        _mlp_body,
        grid=(B * K // MB,),
        in_specs=[n_spec, n_spec, w_spec, v_spec, w_spec, v_spec,
                  w_spec, v_spec, w_spec, v_spec, v_spec, v_spec],
        out_specs=[n_spec, n_spec],
        out_shape=[jax.ShapeDtypeStruct((B * K, D), f32),
                   jax.ShapeDtypeStruct((B * K, D), f32)],
    )(anodes, lnodes,
      W1_0, b1_0.reshape(1, D), W2_0, b2_0.reshape(1, D),
      W1_1, b1_1.reshape(1, D), W2_1, b2_1.reshape(1, D),
      ln_g.reshape(1, D), ln_b.reshape(1, D))

    aref = jax.new_ref(actor_copy.reshape(B * NA, D))
    lref = jax.new_ref(lane_copy.reshape(B * NL, D))
    _sc_scatter(aref, lref, upda, updl, aidx, lidx)
    actor_out = jax.freeze(aref).reshape(B, NA, D)
    lane_out = jax.freeze(lref).reshape(B, NL, D)
    return (actor_out, lane_out)
